# baseline (speedup 1.0000x reference)
import jax
import jax.numpy as jnp
from jax import lax
from jax.experimental import pallas as pl
from jax.experimental.pallas import tpu as pltpu
from jax.experimental.pallas import tpu_sc as plsc

_NUM_CORES = 2
_NUM_SUBCORES = 16
_NUM_WORKERS = _NUM_CORES * _NUM_SUBCORES
_LANES = 16

_BATCH = 4
_SEQ = 8192
_TOTAL = _BATCH * _SEQ
_PER_W = _TOTAL // _NUM_WORKERS
_D = 32
_OUT_W = 2 * _D
_GROUPS = _PER_W // _LANES
_NCHUNK = 8
_GPC = _GROUPS // _NCHUNK
_CHUNK_F = _PER_W * _OUT_W // _NCHUNK


def _sc_body(pos_hbm, w0_hbm, w1_hbm, out_hbm,
             pos_v, w0_v, w1_v, out_v, sem_out):
    wid = lax.axis_index("s") * _NUM_CORES + lax.axis_index("c")
    base = wid * _PER_W

    pltpu.sync_copy(pos_hbm.at[pl.ds(base, _PER_W)], pos_v)
    pltpu.sync_copy(w0_hbm, w0_v)
    pltpu.sync_copy(w1_hbm, w1_v)

    iota = lax.iota(jnp.int32, _LANES)
    iota16 = iota + _LANES
    iota32 = iota + 2 * _LANES
    iota48 = iota + 3 * _LANES

    @pl.loop(0, _NCHUNK)
    def _(k):
        @plsc.parallel_loop(0, _PER_W // _NCHUNK, unroll=4)
        def _(j):
            r = k * (_PER_W // _NCHUNK) + j
            ps = plsc.load_gather(pos_v, [jnp.full((_LANES,), r, jnp.int32)])
            b0 = lax.shift_left(lax.bitwise_and(ps, 255), 5)
            b1 = lax.shift_left(
                lax.bitwise_and(lax.shift_right_logical(ps, 8), 127), 5)
            ob = jnp.full((_LANES,), r * _OUT_W, jnp.int32)
            v00 = plsc.load_gather(w0_v, [b0 + iota])
            plsc.store_scatter(out_v, [ob + iota], v00)
            v01 = plsc.load_gather(w0_v, [b0 + iota16])
            plsc.store_scatter(out_v, [ob + iota16], v01)
            v10 = plsc.load_gather(w1_v, [b1 + iota])
            plsc.store_scatter(out_v, [ob + iota32], v10)
            v11 = plsc.load_gather(w1_v, [b1 + iota16])
            plsc.store_scatter(out_v, [ob + iota48], v11)

        pltpu.async_copy(
            out_v.at[pl.ds(k * _CHUNK_F, _CHUNK_F)],
            out_hbm.at[pl.ds(base * _OUT_W + k * _CHUNK_F, _CHUNK_F)],
            sem_out)

    pltpu.make_async_copy(
        out_v, out_hbm.at[pl.ds(base * _OUT_W, _PER_W * _OUT_W)],
        sem_out).wait()


@jax.jit
def _sc_embed(pos_seq, w0, w1):
    mesh = plsc.VectorSubcoreMesh(core_axis_name="c", subcore_axis_name="s")
    k = pl.kernel(
        _sc_body,
        mesh=mesh,
        compiler_params=pltpu.CompilerParams(
            use_tc_tiling_on_sc=False, needs_layout_passes=False),
        out_type=jax.ShapeDtypeStruct((_TOTAL * _OUT_W,), jnp.float32),
        scratch_types=[
            pltpu.VMEM((_PER_W,), jnp.int32),
            pltpu.VMEM((256 * _D,), jnp.float32),
            pltpu.VMEM((128 * _D,), jnp.float32),
            pltpu.VMEM((_PER_W * _OUT_W,), jnp.float32),
            pltpu.SemaphoreType.DMA,
        ],
    )
    pos_lin = jnp.bitwise_and(pos_seq.reshape(_TOTAL), jnp.int32(32767))
    out = k(pos_lin,
            w0.reshape(256 * _D),
            w1.reshape(128 * _D))
    return out.reshape(_BATCH, _SEQ, _OUT_W)


def kernel(pos_seq, W0, W1):
    return _sc_embed(pos_seq, W0, W1)


# speedup vs baseline: 1.0370x; 1.0370x over previous
import jax
import jax.numpy as jnp
from jax import lax
from jax.experimental import pallas as pl
from jax.experimental.pallas import tpu as pltpu
from jax.experimental.pallas import tpu_sc as plsc

_NUM_CORES = 2
_NUM_SUBCORES = 16
_NUM_WORKERS = _NUM_CORES * _NUM_SUBCORES
_LANES = 16

_BATCH = 4
_SEQ = 8192
_TOTAL = _BATCH * _SEQ
_PER_W = _TOTAL // _NUM_WORKERS
_D = 32
_OUT_W = 2 * _D
_GROUPS = _PER_W // _LANES
_NCHUNK = 8
_GPC = _GROUPS // _NCHUNK
_CHUNK_F = _PER_W * _OUT_W // _NCHUNK


def _sc_body(pos_hbm, w0_hbm, w1_hbm, out_hbm,
             pos_v, w0_v, w1_v, out_v, sem_in, sem_out):
    wid = lax.axis_index("s") * _NUM_CORES + lax.axis_index("c")
    base = wid * _PER_W

    c0 = pltpu.async_copy(pos_hbm.at[pl.ds(base, _PER_W)], pos_v, sem_in)
    c1 = pltpu.async_copy(w0_hbm, w0_v, sem_in)
    c2 = pltpu.async_copy(w1_hbm, w1_v, sem_in)
    c0.wait()
    c1.wait()
    c2.wait()

    iota = lax.iota(jnp.int32, _LANES)
    iota16 = iota + _LANES
    iota32 = iota + 2 * _LANES
    iota48 = iota + 3 * _LANES

    @pl.loop(0, _NCHUNK)
    def _(k):
        @plsc.parallel_loop(0, _PER_W // _NCHUNK, unroll=4)
        def _(j):
            r = k * (_PER_W // _NCHUNK) + j
            ps = plsc.load_gather(pos_v, [jnp.full((_LANES,), r, jnp.int32)])
            b0 = lax.shift_left(lax.bitwise_and(ps, 255), 5)
            b1 = lax.shift_left(
                lax.bitwise_and(lax.shift_right_logical(ps, 8), 127), 5)
            ob = jnp.full((_LANES,), r * _OUT_W, jnp.int32)
            v00 = plsc.load_gather(w0_v, [b0 + iota])
            plsc.store_scatter(out_v, [ob + iota], v00)
            v01 = plsc.load_gather(w0_v, [b0 + iota16])
            plsc.store_scatter(out_v, [ob + iota16], v01)
            v10 = plsc.load_gather(w1_v, [b1 + iota])
            plsc.store_scatter(out_v, [ob + iota32], v10)
            v11 = plsc.load_gather(w1_v, [b1 + iota16])
            plsc.store_scatter(out_v, [ob + iota48], v11)

        pltpu.async_copy(
            out_v.at[pl.ds(k * _CHUNK_F, _CHUNK_F)],
            out_hbm.at[pl.ds(base * _OUT_W + k * _CHUNK_F, _CHUNK_F)],
            sem_out)

    pltpu.make_async_copy(
        out_v, out_hbm.at[pl.ds(base * _OUT_W, _PER_W * _OUT_W)],
        sem_out).wait()


@jax.jit
def _sc_embed(pos_seq, w0, w1):
    mesh = plsc.VectorSubcoreMesh(core_axis_name="c", subcore_axis_name="s")
    k = pl.kernel(
        _sc_body,
        mesh=mesh,
        compiler_params=pltpu.CompilerParams(
            use_tc_tiling_on_sc=False, needs_layout_passes=False),
        out_type=jax.ShapeDtypeStruct((_TOTAL * _OUT_W,), jnp.float32),
        scratch_types=[
            pltpu.VMEM((_PER_W,), jnp.int32),
            pltpu.VMEM((256 * _D,), jnp.float32),
            pltpu.VMEM((128 * _D,), jnp.float32),
            pltpu.VMEM((_PER_W * _OUT_W,), jnp.float32),
            pltpu.SemaphoreType.DMA,
            pltpu.SemaphoreType.DMA,
        ],
    )
    out = k(pos_seq.reshape(_TOTAL),
            w0.reshape(256 * _D),
            w1.reshape(128 * _D))
    return out.reshape(_BATCH, _SEQ, _OUT_W)


def kernel(pos_seq, W0, W1):
    return _sc_embed(pos_seq, W0, W1)
